# baseline (speedup 1.0000x reference)
import functools

import jax
import jax.numpy as jnp
from jax.experimental import pallas as pl
from jax.experimental.pallas import tpu as pltpu

_ROTS = ((13, 15, 26, 6), (17, 29, 16, 24))


def _gumbel_bits(lin):
    k0 = jnp.uint32(0)
    k1 = jnp.uint32(42)
    k2 = k0 ^ k1 ^ jnp.uint32(0x1BD11BDA)
    ks = (k0, k1, k2)
    x0 = jnp.zeros_like(lin)
    x1 = lin + k1
    for i in range(5):
        for r in _ROTS[i % 2]:
            x0 = x0 + x1
            x1 = (x1 << jnp.uint32(r)) | (x1 >> jnp.uint32(32 - r))
            x1 = x0 ^ x1
        x0 = x0 + ks[(i + 1) % 3]
        x1 = x1 + ks[(i + 2) % 3] + jnp.uint32(i + 1)
    return x0 ^ x1


def _gumbel(lin):
    bits = _gumbel_bits(lin)
    u = jax.lax.bitcast_convert_type(
        (bits >> jnp.uint32(9)) | jnp.uint32(0x3F800000), jnp.float32) - 1.0
    return -jnp.log(-jnp.log(u))


def _body(x_ref, samples_ref, lp_ref, acc_y, acc_i, acc_s, *, cb, v):
    i = pl.program_id(0)
    g = pl.num_programs(0)
    b = x_ref.shape[0]

    @pl.when(i == 0)
    def _init():
        acc_y[...] = jnp.full_like(acc_y, -jnp.inf)
        acc_i[...] = jnp.zeros_like(acc_i)
        acc_s[...] = jnp.zeros_like(acc_s)

    roff = jax.lax.broadcasted_iota(jnp.int32, (b, 1), 0) * v
    lane = jax.lax.broadcasted_iota(jnp.int32, (b, 128), 1)
    base_lin = roff + lane

    ay = acc_y[...]
    ai = acc_i[...]
    asum = acc_s[...]
    for j in range(cb // 128):
        x = x_ref[:, j * 128:(j + 1) * 128]
        off = i * cb + j * 128
        lin = base_lin + off
        gum = _gumbel(lin.astype(jnp.uint32))
        valid = lane < (v - off)
        y = jnp.where(valid, x + gum, -jnp.inf)
        upd = y > ay
        ay = jnp.where(upd, y, ay)
        ai = jnp.where(upd, lin, ai)
        asum = asum + jnp.where(valid, jnp.exp(x), 0.0)
    acc_y[...] = ay
    acc_i[...] = ai
    acc_s[...] = asum

    @pl.when(i == g - 1)
    def _finish():
        ayf = acc_y[...]
        m = jnp.max(ayf, axis=1, keepdims=True)
        lin_w = jnp.min(
            jnp.where(ayf == m, acc_i[...], jnp.int32(0x7FFFFFFF)),
            axis=1, keepdims=True)
        gum_w = _gumbel(lin_w.astype(jnp.uint32))
        lse = jnp.log(jnp.sum(acc_s[...], axis=1, keepdims=True))
        samples_ref[...] = lin_w - roff
        lp_ref[...] = (m - gum_w) - lse


def kernel(logits):
    b, v = logits.shape
    cb = 4096
    grid = (pl.cdiv(v, cb),)
    samples, lp = pl.pallas_call(
        functools.partial(_body, cb=cb, v=v),
        grid=grid,
        in_specs=[pl.BlockSpec((b, cb), lambda i: (0, i))],
        out_specs=[pl.BlockSpec((b, 1), lambda i: (0, 0)),
                   pl.BlockSpec((b, 1), lambda i: (0, 0))],
        out_shape=[jax.ShapeDtypeStruct((b, 1), jnp.int32),
                   jax.ShapeDtypeStruct((b, 1), jnp.float32)],
        scratch_shapes=[
            pltpu.VMEM((b, 128), jnp.float32),
            pltpu.VMEM((b, 128), jnp.int32),
            pltpu.VMEM((b, 128), jnp.float32),
        ],
    )(logits)
    return samples[:, 0], lp[:, 0]


# speedup vs baseline: 1.0284x; 1.0284x over previous
import functools

import jax
import jax.numpy as jnp
from jax.experimental import pallas as pl
from jax.experimental.pallas import tpu as pltpu

_ROTS = ((13, 15, 26, 6), (17, 29, 16, 24))


def _gumbel_bits(lin42):
    k0 = jnp.uint32(0)
    k1 = jnp.uint32(42)
    k2 = k0 ^ k1 ^ jnp.uint32(0x1BD11BDA)
    ks = (k0, k1, k2)
    x0 = jnp.zeros_like(lin42)
    x1 = lin42
    for i in range(5):
        for r in _ROTS[i % 2]:
            x0 = x0 + x1
            x1 = (x1 << jnp.uint32(r)) | (x1 >> jnp.uint32(32 - r))
            x1 = x0 ^ x1
        x0 = x0 + ks[(i + 1) % 3]
        x1 = x1 + ks[(i + 2) % 3] + jnp.uint32(i + 1)
    return x0 ^ x1


def _gumbel_pre(lin42):
    bits = _gumbel_bits(lin42)
    u = jax.lax.bitcast_convert_type(
        (bits >> jnp.uint32(9)) | jnp.uint32(0x3F800000), jnp.float32) - 1.0
    return -jnp.log(-jnp.log(u))


def _body(x_ref, samples_ref, lp_ref, acc_y, acc_i, acc_s, *, cb, v):
    i = pl.program_id(0)
    g = pl.num_programs(0)
    b = x_ref.shape[0]

    @pl.when(i == 0)
    def _init():
        acc_y[...] = jnp.full_like(acc_y, -jnp.inf)
        acc_i[...] = jnp.zeros_like(acc_i)
        acc_s[...] = jnp.zeros_like(acc_s)

    roff = jax.lax.broadcasted_iota(jnp.int32, (b, 1), 0) * v
    lane = jax.lax.broadcasted_iota(jnp.int32, (b, 128), 1)
    base42 = roff + lane + 42

    def scan_block(masked):
        ay = acc_y[...]
        ai = acc_i[...]
        asum = acc_s[...]
        for j in range(cb // 128):
            x = x_ref[:, j * 128:(j + 1) * 128]
            off = i * cb + j * 128
            lin42 = base42 + off
            gum = _gumbel_pre(lin42.astype(jnp.uint32))
            if masked:
                valid = lane < (v - off)
                y = jnp.where(valid, x + gum, -jnp.inf)
                es = jnp.where(valid, jnp.exp(x), 0.0)
            else:
                y = x + gum
                es = jnp.exp(x)
            upd = y > ay
            ay = jnp.where(upd, y, ay)
            ai = jnp.where(upd, lin42, ai)
            asum = asum + es
        acc_y[...] = ay
        acc_i[...] = ai
        acc_s[...] = asum

    @pl.when(i < g - 1)
    def _full():
        scan_block(False)

    @pl.when(i == g - 1)
    def _ragged():
        scan_block(True)

    @pl.when(i == g - 1)
    def _finish():
        ayf = acc_y[...]
        m = jnp.max(ayf, axis=1, keepdims=True)
        lin42_w = jnp.min(
            jnp.where(ayf == m, acc_i[...], jnp.int32(0x7FFFFFFF)),
            axis=1, keepdims=True)
        gum_w = _gumbel_pre(lin42_w.astype(jnp.uint32))
        lse = jnp.log(jnp.sum(acc_s[...], axis=1, keepdims=True))
        samples_ref[...] = lin42_w - 42 - roff
        lp_ref[...] = (m - gum_w) - lse


def kernel(logits):
    b, v = logits.shape
    cb = 2048
    grid = (pl.cdiv(v, cb),)
    samples, lp = pl.pallas_call(
        functools.partial(_body, cb=cb, v=v),
        grid=grid,
        in_specs=[pl.BlockSpec((b, cb), lambda i: (0, i))],
        out_specs=[pl.BlockSpec((b, 1), lambda i: (0, 0)),
                   pl.BlockSpec((b, 1), lambda i: (0, 0))],
        out_shape=[jax.ShapeDtypeStruct((b, 1), jnp.int32),
                   jax.ShapeDtypeStruct((b, 1), jnp.float32)],
        scratch_shapes=[
            pltpu.VMEM((b, 128), jnp.float32),
            pltpu.VMEM((b, 128), jnp.int32),
            pltpu.VMEM((b, 128), jnp.float32),
        ],
    )(logits)
    return samples[:, 0], lp[:, 0]


# speedup vs baseline: 1.1037x; 1.0732x over previous
import functools

import jax
import jax.numpy as jnp
from jax import lax
from jax.experimental import pallas as pl
from jax.experimental.pallas import tpu as pltpu
from jax.experimental.pallas import tpu_sc as plsc

_ROTS = ((13, 15, 26, 6), (17, 29, 16, 24))


def _gumbel_bits(lin42):
    k0 = jnp.uint32(0)
    k1 = jnp.uint32(42)
    k2 = k0 ^ k1 ^ jnp.uint32(0x1BD11BDA)
    ks = (k0, k1, k2)
    x0 = jnp.zeros_like(lin42)
    x1 = lin42
    for i in range(5):
        for r in _ROTS[i % 2]:
            x0 = x0 + x1
            x1 = (x1 << jnp.uint32(r)) | (x1 >> jnp.uint32(32 - r))
            x1 = x0 ^ x1
        x0 = x0 + ks[(i + 1) % 3]
        x1 = x1 + ks[(i + 2) % 3] + jnp.uint32(i + 1)
    return x0 ^ x1


def _bits_to_gumbel(bits):
    u = jax.lax.bitcast_convert_type(
        (bits >> jnp.uint32(9)) | jnp.uint32(0x3F800000), jnp.float32) - 1.0
    return -jnp.log(-jnp.log(u))


def _gumbel_pre(lin42):
    return _bits_to_gumbel(_gumbel_bits(lin42))


def _sc_bits_kernel(b, v, c_start, s):
    nw = 32
    rows_per = b // nw
    mesh = plsc.VectorSubcoreMesh(core_axis_name="c", subcore_axis_name="s")

    @functools.partial(
        pl.kernel, mesh=mesh,
        out_type=jax.ShapeDtypeStruct((b, s), jnp.uint32),
        scratch_types=[pltpu.VMEM((s,), jnp.uint32)],
    )
    def gen(out_hbm, buf):
        wid = lax.axis_index("c") * 16 + lax.axis_index("s")
        vec_iota = lax.iota(jnp.uint32, 16)
        for rr in range(rows_per):
            row = wid * rows_per + rr
            base42 = (row * v + (c_start + 42)).astype(jnp.uint32)

            def vec_body(j, _):
                o = j * 64
                for k in range(4):
                    lin42 = base42 + (jnp.uint32(o + k * 16) + vec_iota)
                    buf[pl.ds(o + k * 16, 16)] = _gumbel_bits(lin42)
                return 0

            lax.fori_loop(0, s // 64, vec_body, 0)
            pltpu.sync_copy(buf, out_hbm.at[row])

    return gen()


def _tc_main_body(x_ref, ay_ref, ai_ref, as_ref, acc_y, acc_i, acc_s, *,
                  cb, v):
    i = pl.program_id(0)
    g = pl.num_programs(0)
    b = x_ref.shape[0]

    @pl.when(i == 0)
    def _init():
        acc_y[...] = jnp.full_like(acc_y, -jnp.inf)
        acc_i[...] = jnp.zeros_like(acc_i)
        acc_s[...] = jnp.zeros_like(acc_s)

    roff = jax.lax.broadcasted_iota(jnp.int32, (b, 1), 0) * v
    lane = jax.lax.broadcasted_iota(jnp.int32, (b, 128), 1)
    base42 = roff + lane + 42

    ay = acc_y[...]
    ai = acc_i[...]
    asum = acc_s[...]
    for j in range(cb // 128):
        x = x_ref[:, j * 128:(j + 1) * 128]
        off = i * cb + j * 128
        lin42 = base42 + off
        gum = _bits_to_gumbel(_gumbel_bits(lin42.astype(jnp.uint32)))
        y = x + gum
        upd = y > ay
        ay = jnp.where(upd, y, ay)
        ai = jnp.where(upd, lin42, ai)
        asum = asum + jnp.exp(x)
    acc_y[...] = ay
    acc_i[...] = ai
    acc_s[...] = asum

    @pl.when(i == g - 1)
    def _emit():
        ay_ref[...] = acc_y[...]
        ai_ref[...] = acc_i[...]
        as_ref[...] = acc_s[...]


def _tc_tail_body(x_ref, bits_ref, ay0_ref, ai0_ref, as0_ref,
                  samples_ref, lp_ref, acc_y, acc_i, acc_s, *,
                  cb, v, c_start, s):
    i = pl.program_id(0)
    g = pl.num_programs(0)
    b = x_ref.shape[0]

    @pl.when(i == 0)
    def _init():
        acc_y[...] = ay0_ref[...]
        acc_i[...] = ai0_ref[...]
        acc_s[...] = as0_ref[...]

    roff = jax.lax.broadcasted_iota(jnp.int32, (b, 1), 0) * v
    lane = jax.lax.broadcasted_iota(jnp.int32, (b, 128), 1)
    base42 = roff + lane + (c_start + 42)

    def scan_block(masked):
        ay = acc_y[...]
        ai = acc_i[...]
        asum = acc_s[...]
        for j in range(cb // 128):
            x = x_ref[:, j * 128:(j + 1) * 128]
            bits = bits_ref[:, j * 128:(j + 1) * 128]
            off = i * cb + j * 128
            lin42 = base42 + off
            gum = _bits_to_gumbel(bits)
            if masked:
                valid = lane < (s - off)
                y = jnp.where(valid, x + gum, -jnp.inf)
                es = jnp.where(valid, jnp.exp(x), 0.0)
            else:
                y = x + gum
                es = jnp.exp(x)
            upd = y > ay
            ay = jnp.where(upd, y, ay)
            ai = jnp.where(upd, lin42, ai)
            asum = asum + es
        acc_y[...] = ay
        acc_i[...] = ai
        acc_s[...] = asum

    @pl.when(i < g - 1)
    def _full():
        scan_block(False)

    @pl.when(i == g - 1)
    def _ragged():
        scan_block(True)

    @pl.when(i == g - 1)
    def _finish():
        ayf = acc_y[...]
        m = jnp.max(ayf, axis=1, keepdims=True)
        lin42_w = jnp.min(
            jnp.where(ayf == m, acc_i[...], jnp.int32(0x7FFFFFFF)),
            axis=1, keepdims=True)
        gum_w = _gumbel_pre(lin42_w.astype(jnp.uint32))
        lse = jnp.log(jnp.sum(acc_s[...], axis=1, keepdims=True))
        samples_ref[...] = lin42_w - 42 - roff
        lp_ref[...] = (m - gum_w) - lse


def kernel(logits):
    b, v = logits.shape
    cb = 2048
    nmain = max(v // cb - 60, 0)
    c_start = nmain * cb
    s = v - c_start

    bits = _sc_bits_kernel(b, v, c_start, s)

    ay, ai, asum = pl.pallas_call(
        functools.partial(_tc_main_body, cb=cb, v=v),
        grid=(nmain,),
        in_specs=[pl.BlockSpec((b, cb), lambda i: (0, i))],
        out_specs=[pl.BlockSpec((b, 128), lambda i: (0, 0)),
                   pl.BlockSpec((b, 128), lambda i: (0, 0)),
                   pl.BlockSpec((b, 128), lambda i: (0, 0))],
        out_shape=[jax.ShapeDtypeStruct((b, 128), jnp.float32),
                   jax.ShapeDtypeStruct((b, 128), jnp.int32),
                   jax.ShapeDtypeStruct((b, 128), jnp.float32)],
        scratch_shapes=[
            pltpu.VMEM((b, 128), jnp.float32),
            pltpu.VMEM((b, 128), jnp.int32),
            pltpu.VMEM((b, 128), jnp.float32),
        ],
    )(logits)

    samples, lp = pl.pallas_call(
        functools.partial(_tc_tail_body, cb=cb, v=v, c_start=c_start, s=s),
        grid=(pl.cdiv(s, cb),),
        in_specs=[pl.BlockSpec((b, cb), lambda i: (0, i + nmain)),
                  pl.BlockSpec((b, cb), lambda i: (0, i)),
                  pl.BlockSpec((b, 128), lambda i: (0, 0)),
                  pl.BlockSpec((b, 128), lambda i: (0, 0)),
                  pl.BlockSpec((b, 128), lambda i: (0, 0))],
        out_specs=[pl.BlockSpec((b, 1), lambda i: (0, 0)),
                   pl.BlockSpec((b, 1), lambda i: (0, 0))],
        out_shape=[jax.ShapeDtypeStruct((b, 1), jnp.int32),
                   jax.ShapeDtypeStruct((b, 1), jnp.float32)],
        scratch_shapes=[
            pltpu.VMEM((b, 128), jnp.float32),
            pltpu.VMEM((b, 128), jnp.int32),
            pltpu.VMEM((b, 128), jnp.float32),
        ],
    )(logits, bits, ay, ai, asum)
    return samples[:, 0], lp[:, 0]


# speedup vs baseline: 1.1965x; 1.0841x over previous
import functools

import jax
import jax.numpy as jnp
from jax import lax
from jax.experimental import pallas as pl
from jax.experimental.pallas import tpu as pltpu
from jax.experimental.pallas import tpu_sc as plsc

_ROTS = ((13, 15, 26, 6), (17, 29, 16, 24))


def _gumbel_bits(lin42):
    k0 = jnp.uint32(0)
    k1 = jnp.uint32(42)
    k2 = k0 ^ k1 ^ jnp.uint32(0x1BD11BDA)
    ks = (k0, k1, k2)
    x0 = jnp.zeros_like(lin42)
    x1 = lin42
    for i in range(5):
        for r in _ROTS[i % 2]:
            x0 = x0 + x1
            x1 = (x1 << jnp.uint32(r)) | (x1 >> jnp.uint32(32 - r))
            x1 = x0 ^ x1
        x0 = x0 + ks[(i + 1) % 3]
        x1 = x1 + ks[(i + 2) % 3] + jnp.uint32(i + 1)
    return x0 ^ x1


def _bits_to_gumbel(bits):
    u = jax.lax.bitcast_convert_type(
        (bits >> jnp.uint32(9)) | jnp.uint32(0x3F800000), jnp.float32) - 1.0
    return -jnp.log(-jnp.log(u))


def _gumbel_pre(lin42):
    return _bits_to_gumbel(_gumbel_bits(lin42))


def _sc_bits_kernel(b, v, c_start, s):
    nw = 32
    rows_per = b // nw
    mesh = plsc.VectorSubcoreMesh(core_axis_name="c", subcore_axis_name="s")

    chunk = 16384
    nchunk = -(-s // chunk)
    sbits = nchunk * chunk

    @functools.partial(
        pl.kernel, mesh=mesh,
        out_type=jax.ShapeDtypeStruct((b, sbits), jnp.uint32),
        scratch_types=[pltpu.VMEM((chunk,), jnp.uint32)],
    )
    def gen(out_hbm, buf):
        wid = lax.axis_index("c") * 16 + lax.axis_index("s")
        vec_iota = lax.iota(jnp.uint32, 16)
        for rr in range(rows_per):
            row = wid * rows_per + rr
            base42 = (row * v + (c_start + 42)).astype(jnp.uint32)

            def chunk_body(ch, _):
                coff = ch * chunk
                cbase42 = base42 + coff.astype(jnp.uint32)

                def vec_body(j, _):
                    o = j * 64
                    for k in range(4):
                        lin42 = cbase42 + (jnp.uint32(o + k * 16) + vec_iota)
                        buf[pl.ds(o + k * 16, 16)] = _gumbel_bits(lin42)
                    return 0

                lax.fori_loop(0, chunk // 64, vec_body, 0)
                pltpu.sync_copy(buf, out_hbm.at[row, pl.ds(coff, chunk)])
                return 0

            lax.fori_loop(0, nchunk, chunk_body, 0)

    return gen()


def _tc_main_body(x_ref, ay_ref, ai_ref, as_ref, acc_y, acc_i, acc_s, *,
                  cb, v):
    i = pl.program_id(0)
    g = pl.num_programs(0)
    b = x_ref.shape[0]

    @pl.when(i == 0)
    def _init():
        acc_y[...] = jnp.full_like(acc_y, -jnp.inf)
        acc_i[...] = jnp.zeros_like(acc_i)
        acc_s[...] = jnp.zeros_like(acc_s)

    roff = jax.lax.broadcasted_iota(jnp.int32, (b, 1), 0) * v
    lane = jax.lax.broadcasted_iota(jnp.int32, (b, 128), 1)
    base42 = roff + lane + 42

    ay = acc_y[...]
    ai = acc_i[...]
    asum = acc_s[...]
    for j in range(cb // 128):
        x = x_ref[:, j * 128:(j + 1) * 128]
        off = i * cb + j * 128
        lin42 = base42 + off
        gum = _bits_to_gumbel(_gumbel_bits(lin42.astype(jnp.uint32)))
        y = x + gum
        upd = y > ay
        ay = jnp.where(upd, y, ay)
        ai = jnp.where(upd, lin42, ai)
        asum = asum + jnp.exp(x)
    acc_y[...] = ay
    acc_i[...] = ai
    acc_s[...] = asum

    @pl.when(i == g - 1)
    def _emit():
        ay_ref[...] = acc_y[...]
        ai_ref[...] = acc_i[...]
        as_ref[...] = acc_s[...]


def _tc_tail_body(x_ref, bits_ref, ay0_ref, ai0_ref, as0_ref,
                  samples_ref, lp_ref, acc_y, acc_i, acc_s, *,
                  cb, v, c_start, s):
    i = pl.program_id(0)
    g = pl.num_programs(0)
    b = x_ref.shape[0]

    @pl.when(i == 0)
    def _init():
        acc_y[...] = ay0_ref[...]
        acc_i[...] = ai0_ref[...]
        acc_s[...] = as0_ref[...]

    roff = jax.lax.broadcasted_iota(jnp.int32, (b, 1), 0) * v
    lane = jax.lax.broadcasted_iota(jnp.int32, (b, 128), 1)
    base42 = roff + lane + (c_start + 42)

    def scan_block(masked):
        ay = acc_y[...]
        ai = acc_i[...]
        asum = acc_s[...]
        for j in range(cb // 128):
            x = x_ref[:, j * 128:(j + 1) * 128]
            bits = bits_ref[:, j * 128:(j + 1) * 128]
            off = i * cb + j * 128
            lin42 = base42 + off
            gum = _bits_to_gumbel(bits)
            if masked:
                valid = lane < (s - off)
                y = jnp.where(valid, x + gum, -jnp.inf)
                es = jnp.where(valid, jnp.exp(x), 0.0)
            else:
                y = x + gum
                es = jnp.exp(x)
            upd = y > ay
            ay = jnp.where(upd, y, ay)
            ai = jnp.where(upd, lin42, ai)
            asum = asum + es
        acc_y[...] = ay
        acc_i[...] = ai
        acc_s[...] = asum

    @pl.when(i < g - 1)
    def _full():
        scan_block(False)

    @pl.when(i == g - 1)
    def _ragged():
        scan_block(True)

    @pl.when(i == g - 1)
    def _finish():
        ayf = acc_y[...]
        m = jnp.max(ayf, axis=1, keepdims=True)
        lin42_w = jnp.min(
            jnp.where(ayf == m, acc_i[...], jnp.int32(0x7FFFFFFF)),
            axis=1, keepdims=True)
        gum_w = _gumbel_pre(lin42_w.astype(jnp.uint32))
        lse = jnp.log(jnp.sum(acc_s[...], axis=1, keepdims=True))
        samples_ref[...] = lin42_w - 42 - roff
        lp_ref[...] = (m - gum_w) - lse


def kernel(logits):
    b, v = logits.shape
    cb = 2048
    nmain = max(v // cb - 136, 0)
    c_start = nmain * cb
    s = v - c_start

    bits = _sc_bits_kernel(b, v, c_start, s)

    ay, ai, asum = pl.pallas_call(
        functools.partial(_tc_main_body, cb=cb, v=v),
        grid=(nmain,),
        in_specs=[pl.BlockSpec((b, cb), lambda i: (0, i))],
        out_specs=[pl.BlockSpec((b, 128), lambda i: (0, 0)),
                   pl.BlockSpec((b, 128), lambda i: (0, 0)),
                   pl.BlockSpec((b, 128), lambda i: (0, 0))],
        out_shape=[jax.ShapeDtypeStruct((b, 128), jnp.float32),
                   jax.ShapeDtypeStruct((b, 128), jnp.int32),
                   jax.ShapeDtypeStruct((b, 128), jnp.float32)],
        scratch_shapes=[
            pltpu.VMEM((b, 128), jnp.float32),
            pltpu.VMEM((b, 128), jnp.int32),
            pltpu.VMEM((b, 128), jnp.float32),
        ],
    )(logits)

    samples, lp = pl.pallas_call(
        functools.partial(_tc_tail_body, cb=cb, v=v, c_start=c_start, s=s),
        grid=(pl.cdiv(s, cb),),
        in_specs=[pl.BlockSpec((b, cb), lambda i: (0, i + nmain)),
                  pl.BlockSpec((b, cb), lambda i: (0, i)),
                  pl.BlockSpec((b, 128), lambda i: (0, 0)),
                  pl.BlockSpec((b, 128), lambda i: (0, 0)),
                  pl.BlockSpec((b, 128), lambda i: (0, 0))],
        out_specs=[pl.BlockSpec((b, 1), lambda i: (0, 0)),
                   pl.BlockSpec((b, 1), lambda i: (0, 0))],
        out_shape=[jax.ShapeDtypeStruct((b, 1), jnp.int32),
                   jax.ShapeDtypeStruct((b, 1), jnp.float32)],
        scratch_shapes=[
            pltpu.VMEM((b, 128), jnp.float32),
            pltpu.VMEM((b, 128), jnp.int32),
            pltpu.VMEM((b, 128), jnp.float32),
        ],
    )(logits, bits, ay, ai, asum)
    return samples[:, 0], lp[:, 0]


# speedup vs baseline: 1.2147x; 1.0152x over previous
import functools

import jax
import jax.numpy as jnp
from jax import lax
from jax.experimental import pallas as pl
from jax.experimental.pallas import tpu as pltpu
from jax.experimental.pallas import tpu_sc as plsc

_ROTS = ((13, 15, 26, 6), (17, 29, 16, 24))


def _gumbel_bits(lin42):
    k0 = jnp.uint32(0)
    k1 = jnp.uint32(42)
    k2 = k0 ^ k1 ^ jnp.uint32(0x1BD11BDA)
    ks = (k0, k1, k2)
    x0 = jnp.zeros_like(lin42)
    x1 = lin42
    for i in range(5):
        for r in _ROTS[i % 2]:
            x0 = x0 + x1
            x1 = (x1 << jnp.uint32(r)) | (x1 >> jnp.uint32(32 - r))
            x1 = x0 ^ x1
        x0 = x0 + ks[(i + 1) % 3]
        x1 = x1 + ks[(i + 2) % 3] + jnp.uint32(i + 1)
    return x0 ^ x1


def _bits_to_gumbel(bits):
    u = jax.lax.bitcast_convert_type(
        (bits >> jnp.uint32(9)) | jnp.uint32(0x3F800000), jnp.float32) - 1.0
    return -jnp.log(-jnp.log(u))


def _gumbel_pre(lin42):
    return _bits_to_gumbel(_gumbel_bits(lin42))


def _sc_bits_kernel(b, v, c_start, s):
    nw = 32
    rows_per = b // nw
    mesh = plsc.VectorSubcoreMesh(core_axis_name="c", subcore_axis_name="s")

    chunk = 16384
    nchunk = -(-s // chunk)
    sbits = nchunk * chunk

    @functools.partial(
        pl.kernel, mesh=mesh,
        out_type=jax.ShapeDtypeStruct((b, sbits), jnp.uint32),
        scratch_types=[pltpu.VMEM((chunk,), jnp.uint32)],
    )
    def gen(out_hbm, buf):
        wid = lax.axis_index("c") * 16 + lax.axis_index("s")
        vec_iota = lax.iota(jnp.uint32, 16)
        for rr in range(rows_per):
            row = wid * rows_per + rr
            base42 = (row * v + (c_start + 42)).astype(jnp.uint32)

            def chunk_body(ch, _):
                coff = ch * chunk
                cbase42 = base42 + coff.astype(jnp.uint32)

                def vec_body(j, _):
                    o = j * 64
                    for k in range(4):
                        lin42 = cbase42 + (jnp.uint32(o + k * 16) + vec_iota)
                        buf[pl.ds(o + k * 16, 16)] = _gumbel_bits(lin42)
                    return 0

                lax.fori_loop(0, chunk // 64, vec_body, 0)
                pltpu.sync_copy(buf, out_hbm.at[row, pl.ds(coff, chunk)])
                return 0

            lax.fori_loop(0, nchunk, chunk_body, 0)

    return gen()


def _tc_main_body(x_ref, ay_ref, ai_ref, as_ref, acc_y, acc_i, acc_s, *,
                  cb, v):
    i = pl.program_id(0)
    g = pl.num_programs(0)
    b = x_ref.shape[0]

    @pl.when(i == 0)
    def _init():
        acc_y[...] = jnp.full_like(acc_y, -jnp.inf)
        acc_i[...] = jnp.zeros_like(acc_i)
        acc_s[...] = jnp.zeros_like(acc_s)

    roff = jax.lax.broadcasted_iota(jnp.int32, (b, 1), 0) * v
    lane = jax.lax.broadcasted_iota(jnp.int32, (b, 128), 1)
    base42 = roff + lane + 42

    ay = acc_y[...]
    ai = acc_i[...]
    asum = acc_s[...]
    for j in range(cb // 128):
        x = x_ref[:, j * 128:(j + 1) * 128]
        off = i * cb + j * 128
        lin42 = base42 + off
        gum = _bits_to_gumbel(_gumbel_bits(lin42.astype(jnp.uint32)))
        y = x + gum
        upd = y > ay
        ay = jnp.where(upd, y, ay)
        ai = jnp.where(upd, lin42, ai)
        asum = asum + jnp.exp(x)
    acc_y[...] = ay
    acc_i[...] = ai
    acc_s[...] = asum

    @pl.when(i == g - 1)
    def _emit():
        ay_ref[...] = acc_y[...]
        ai_ref[...] = acc_i[...]
        as_ref[...] = acc_s[...]


def _tc_tail_body(x_ref, bits_ref, ay0_ref, ai0_ref, as0_ref,
                  samples_ref, lp_ref, acc_y, acc_i, acc_s, *,
                  cb, v, c_start, s):
    i = pl.program_id(0)
    g = pl.num_programs(0)
    b = x_ref.shape[0]

    @pl.when(i == 0)
    def _init():
        acc_y[...] = ay0_ref[...]
        acc_i[...] = ai0_ref[...]
        acc_s[...] = as0_ref[...]

    roff = jax.lax.broadcasted_iota(jnp.int32, (b, 1), 0) * v
    lane = jax.lax.broadcasted_iota(jnp.int32, (b, 128), 1)
    base42 = roff + lane + (c_start + 42)

    def scan_block(masked):
        ay = acc_y[...]
        ai = acc_i[...]
        asum = acc_s[...]
        for j in range(cb // 128):
            x = x_ref[:, j * 128:(j + 1) * 128]
            bits = bits_ref[:, j * 128:(j + 1) * 128]
            off = i * cb + j * 128
            lin42 = base42 + off
            gum = _bits_to_gumbel(bits)
            if masked:
                valid = lane < (s - off)
                y = jnp.where(valid, x + gum, -jnp.inf)
                es = jnp.where(valid, jnp.exp(x), 0.0)
            else:
                y = x + gum
                es = jnp.exp(x)
            upd = y > ay
            ay = jnp.where(upd, y, ay)
            ai = jnp.where(upd, lin42, ai)
            asum = asum + es
        acc_y[...] = ay
        acc_i[...] = ai
        acc_s[...] = asum

    @pl.when(i < g - 1)
    def _full():
        scan_block(False)

    @pl.when(i == g - 1)
    def _ragged():
        scan_block(True)

    @pl.when(i == g - 1)
    def _finish():
        ayf = acc_y[...]
        m = jnp.max(ayf, axis=1, keepdims=True)
        lin42_w = jnp.min(
            jnp.where(ayf == m, acc_i[...], jnp.int32(0x7FFFFFFF)),
            axis=1, keepdims=True)
        gum_w = _gumbel_pre(lin42_w.astype(jnp.uint32))
        lse = jnp.log(jnp.sum(acc_s[...], axis=1, keepdims=True))
        samples_ref[...] = lin42_w - 42 - roff
        lp_ref[...] = (m - gum_w) - lse


def kernel(logits):
    b, v = logits.shape
    cb = 2048
    nmain = max(v // cb - 132, 0)
    c_start = nmain * cb
    s = v - c_start

    bits = _sc_bits_kernel(b, v, c_start, s)

    ay, ai, asum = pl.pallas_call(
        functools.partial(_tc_main_body, cb=cb, v=v),
        grid=(nmain,),
        in_specs=[pl.BlockSpec((b, cb), lambda i: (0, i))],
        out_specs=[pl.BlockSpec((b, 128), lambda i: (0, 0)),
                   pl.BlockSpec((b, 128), lambda i: (0, 0)),
                   pl.BlockSpec((b, 128), lambda i: (0, 0))],
        out_shape=[jax.ShapeDtypeStruct((b, 128), jnp.float32),
                   jax.ShapeDtypeStruct((b, 128), jnp.int32),
                   jax.ShapeDtypeStruct((b, 128), jnp.float32)],
        scratch_shapes=[
            pltpu.VMEM((b, 128), jnp.float32),
            pltpu.VMEM((b, 128), jnp.int32),
            pltpu.VMEM((b, 128), jnp.float32),
        ],
    )(logits)

    samples, lp = pl.pallas_call(
        functools.partial(_tc_tail_body, cb=cb, v=v, c_start=c_start, s=s),
        grid=(pl.cdiv(s, cb),),
        in_specs=[pl.BlockSpec((b, cb), lambda i: (0, i + nmain)),
                  pl.BlockSpec((b, cb), lambda i: (0, i)),
                  pl.BlockSpec((b, 128), lambda i: (0, 0)),
                  pl.BlockSpec((b, 128), lambda i: (0, 0)),
                  pl.BlockSpec((b, 128), lambda i: (0, 0))],
        out_specs=[pl.BlockSpec((b, 1), lambda i: (0, 0)),
                   pl.BlockSpec((b, 1), lambda i: (0, 0))],
        out_shape=[jax.ShapeDtypeStruct((b, 1), jnp.int32),
                   jax.ShapeDtypeStruct((b, 1), jnp.float32)],
        scratch_shapes=[
            pltpu.VMEM((b, 128), jnp.float32),
            pltpu.VMEM((b, 128), jnp.int32),
            pltpu.VMEM((b, 128), jnp.float32),
        ],
    )(logits, bits, ay, ai, asum)
    return samples[:, 0], lp[:, 0]


# speedup vs baseline: 1.2189x; 1.0034x over previous
import functools

import jax
import jax.numpy as jnp
from jax import lax
from jax.experimental import pallas as pl
from jax.experimental.pallas import tpu as pltpu
from jax.experimental.pallas import tpu_sc as plsc

_ROTS = ((13, 15, 26, 6), (17, 29, 16, 24))


def _gumbel_bits(lin42):
    k0 = jnp.uint32(0)
    k1 = jnp.uint32(42)
    k2 = k0 ^ k1 ^ jnp.uint32(0x1BD11BDA)
    ks = (k0, k1, k2)
    x0 = jnp.zeros_like(lin42)
    x1 = lin42
    for i in range(5):
        for r in _ROTS[i % 2]:
            x0 = x0 + x1
            x1 = (x1 << jnp.uint32(r)) | (x1 >> jnp.uint32(32 - r))
            x1 = x0 ^ x1
        x0 = x0 + ks[(i + 1) % 3]
        x1 = x1 + ks[(i + 2) % 3] + jnp.uint32(i + 1)
    return x0 ^ x1


def _bits_to_gumbel(bits):
    u = jax.lax.bitcast_convert_type(
        (bits >> jnp.uint32(9)) | jnp.uint32(0x3F800000), jnp.float32) - 1.0
    return -jnp.log(-jnp.log(u))


def _gumbel_pre(lin42):
    return _bits_to_gumbel(_gumbel_bits(lin42))


def _sc_bits_kernel(b, v, c_start, s):
    nw = 32
    rows_per = b // nw
    mesh = plsc.VectorSubcoreMesh(core_axis_name="c", subcore_axis_name="s")

    chunk = 16384
    nchunk = -(-s // chunk)
    sbits = nchunk * chunk

    @functools.partial(
        pl.kernel, mesh=mesh,
        out_type=jax.ShapeDtypeStruct((b, sbits), jnp.uint32),
        scratch_types=[pltpu.VMEM((chunk,), jnp.uint32)],
    )
    def gen(out_hbm, buf):
        wid = lax.axis_index("c") * 16 + lax.axis_index("s")
        vec_iota = lax.iota(jnp.uint32, 16)
        for rr in range(rows_per):
            row = wid * rows_per + rr
            base42 = (row * v + (c_start + 42)).astype(jnp.uint32)

            def chunk_body(ch, _):
                coff = ch * chunk
                cbase42 = base42 + coff.astype(jnp.uint32)

                def vec_body(j, _):
                    o = j * 64
                    for k in range(4):
                        lin42 = cbase42 + (jnp.uint32(o + k * 16) + vec_iota)
                        buf[pl.ds(o + k * 16, 16)] = _gumbel_bits(lin42)
                    return 0

                lax.fori_loop(0, chunk // 64, vec_body, 0)
                pltpu.sync_copy(buf, out_hbm.at[row, pl.ds(coff, chunk)])
                return 0

            lax.fori_loop(0, nchunk, chunk_body, 0)

    return gen()


def _tc_main_body(x_ref, ay_ref, ai_ref, as_ref, acc_y, acc_i, acc_s, *,
                  cb, v):
    i = pl.program_id(0)
    g = pl.num_programs(0)
    b = x_ref.shape[0]

    @pl.when(i == 0)
    def _init():
        acc_y[...] = jnp.full_like(acc_y, -jnp.inf)
        acc_i[...] = jnp.zeros_like(acc_i)
        acc_s[...] = jnp.zeros_like(acc_s)

    roff = jax.lax.broadcasted_iota(jnp.int32, (b, 1), 0) * v
    lane = jax.lax.broadcasted_iota(jnp.int32, (b, 128), 1)
    base42 = roff + lane + 42

    ay = acc_y[...]
    ai = acc_i[...]
    asum = acc_s[...]
    for j in range(cb // 128):
        x = x_ref[:, j * 128:(j + 1) * 128]
        off = i * cb + j * 128
        lin42 = base42 + off
        gum = _bits_to_gumbel(_gumbel_bits(lin42.astype(jnp.uint32)))
        y = x + gum
        upd = y > ay
        ay = jnp.where(upd, y, ay)
        ai = jnp.where(upd, lin42, ai)
        asum = asum + jnp.exp(x)
    acc_y[...] = ay
    acc_i[...] = ai
    acc_s[...] = asum

    @pl.when(i == g - 1)
    def _emit():
        ay_ref[...] = acc_y[...]
        ai_ref[...] = acc_i[...]
        as_ref[...] = acc_s[...]


def _tc_tail_body(x_ref, bits_ref, ay0_ref, ai0_ref, as0_ref,
                  samples_ref, lp_ref, acc_y, acc_i, acc_s, *,
                  cb, v, c_start, s):
    i = pl.program_id(0)
    g = pl.num_programs(0)
    b = x_ref.shape[0]

    @pl.when(i == 0)
    def _init():
        acc_y[...] = ay0_ref[...]
        acc_i[...] = ai0_ref[...]
        acc_s[...] = as0_ref[...]

    roff = jax.lax.broadcasted_iota(jnp.int32, (b, 1), 0) * v
    lane = jax.lax.broadcasted_iota(jnp.int32, (b, 128), 1)
    base42 = roff + lane + (c_start + 42)

    def scan_block(masked):
        ay = acc_y[...]
        ai = acc_i[...]
        asum = acc_s[...]
        for j in range(cb // 128):
            x = x_ref[:, j * 128:(j + 1) * 128]
            bits = bits_ref[:, j * 128:(j + 1) * 128]
            off = i * cb + j * 128
            lin42 = base42 + off
            gum = _bits_to_gumbel(bits)
            if masked:
                valid = lane < (s - off)
                y = jnp.where(valid, x + gum, -jnp.inf)
                es = jnp.where(valid, jnp.exp(x), 0.0)
            else:
                y = x + gum
                es = jnp.exp(x)
            upd = y > ay
            ay = jnp.where(upd, y, ay)
            ai = jnp.where(upd, lin42, ai)
            asum = asum + es
        acc_y[...] = ay
        acc_i[...] = ai
        acc_s[...] = asum

    @pl.when(i < g - 1)
    def _full():
        scan_block(False)

    @pl.when(i == g - 1)
    def _ragged():
        scan_block(True)

    @pl.when(i == g - 1)
    def _finish():
        ayf = acc_y[...]
        m = jnp.max(ayf, axis=1, keepdims=True)
        lin42_w = jnp.min(
            jnp.where(ayf == m, acc_i[...], jnp.int32(0x7FFFFFFF)),
            axis=1, keepdims=True)
        gum_w = _gumbel_pre(lin42_w.astype(jnp.uint32))
        lse = jnp.log(jnp.sum(acc_s[...], axis=1, keepdims=True))
        samples_ref[...] = lin42_w - 42 - roff
        lp_ref[...] = (m - gum_w) - lse


def kernel(logits):
    b, v = logits.shape
    cb = 2048
    nmain = max(v // cb - 134, 0)
    c_start = nmain * cb
    s = v - c_start

    bits = _sc_bits_kernel(b, v, c_start, s)

    ay, ai, asum = pl.pallas_call(
        functools.partial(_tc_main_body, cb=cb, v=v),
        grid=(nmain,),
        in_specs=[pl.BlockSpec((b, cb), lambda i: (0, i))],
        out_specs=[pl.BlockSpec((b, 128), lambda i: (0, 0)),
                   pl.BlockSpec((b, 128), lambda i: (0, 0)),
                   pl.BlockSpec((b, 128), lambda i: (0, 0))],
        out_shape=[jax.ShapeDtypeStruct((b, 128), jnp.float32),
                   jax.ShapeDtypeStruct((b, 128), jnp.int32),
                   jax.ShapeDtypeStruct((b, 128), jnp.float32)],
        scratch_shapes=[
            pltpu.VMEM((b, 128), jnp.float32),
            pltpu.VMEM((b, 128), jnp.int32),
            pltpu.VMEM((b, 128), jnp.float32),
        ],
    )(logits)

    samples, lp = pl.pallas_call(
        functools.partial(_tc_tail_body, cb=cb, v=v, c_start=c_start, s=s),
        grid=(pl.cdiv(s, cb),),
        in_specs=[pl.BlockSpec((b, cb), lambda i: (0, i + nmain)),
                  pl.BlockSpec((b, cb), lambda i: (0, i)),
                  pl.BlockSpec((b, 128), lambda i: (0, 0)),
                  pl.BlockSpec((b, 128), lambda i: (0, 0)),
                  pl.BlockSpec((b, 128), lambda i: (0, 0))],
        out_specs=[pl.BlockSpec((b, 1), lambda i: (0, 0)),
                   pl.BlockSpec((b, 1), lambda i: (0, 0))],
        out_shape=[jax.ShapeDtypeStruct((b, 1), jnp.int32),
                   jax.ShapeDtypeStruct((b, 1), jnp.float32)],
        scratch_shapes=[
            pltpu.VMEM((b, 128), jnp.float32),
            pltpu.VMEM((b, 128), jnp.int32),
            pltpu.VMEM((b, 128), jnp.float32),
        ],
    )(logits, bits, ay, ai, asum)
    return samples[:, 0], lp[:, 0]


# speedup vs baseline: 1.2204x; 1.0013x over previous
import functools

import jax
import jax.numpy as jnp
from jax import lax
from jax.experimental import pallas as pl
from jax.experimental.pallas import tpu as pltpu
from jax.experimental.pallas import tpu_sc as plsc

_ROTS = ((13, 15, 26, 6), (17, 29, 16, 24))


def _gumbel_bits(lin42):
    k0 = jnp.uint32(0)
    k1 = jnp.uint32(42)
    k2 = k0 ^ k1 ^ jnp.uint32(0x1BD11BDA)
    ks = (k0, k1, k2)
    x0 = jnp.zeros_like(lin42)
    x1 = lin42
    for i in range(5):
        for r in _ROTS[i % 2]:
            x0 = x0 + x1
            x1 = (x1 << jnp.uint32(r)) | (x1 >> jnp.uint32(32 - r))
            x1 = x0 ^ x1
        x0 = x0 + ks[(i + 1) % 3]
        x1 = x1 + ks[(i + 2) % 3] + jnp.uint32(i + 1)
    return x0 ^ x1


def _bits_to_gumbel(bits):
    u = jax.lax.bitcast_convert_type(
        (bits >> jnp.uint32(9)) | jnp.uint32(0x3F800000), jnp.float32) - 1.0
    return -jnp.log(-jnp.log(u))


def _gumbel_pre(lin42):
    return _bits_to_gumbel(_gumbel_bits(lin42))


def _sc_bits_kernel(b, v, c_start, s):
    nw = 32
    rows_per = b // nw
    mesh = plsc.VectorSubcoreMesh(core_axis_name="c", subcore_axis_name="s")

    chunk = 16384
    nchunk = -(-s // chunk)
    sbits = nchunk * chunk

    @functools.partial(
        pl.kernel, mesh=mesh,
        out_type=jax.ShapeDtypeStruct((b, sbits), jnp.uint32),
        scratch_types=[pltpu.VMEM((chunk,), jnp.uint32)],
    )
    def gen(out_hbm, buf):
        wid = lax.axis_index("c") * 16 + lax.axis_index("s")
        vec_iota = lax.iota(jnp.uint32, 16)
        for rr in range(rows_per):
            row = wid * rows_per + rr
            base42 = (row * v + (c_start + 42)).astype(jnp.uint32)

            def chunk_body(ch, _):
                coff = ch * chunk
                cbase42 = base42 + coff.astype(jnp.uint32)

                def vec_body(j, _):
                    o = j * 64
                    for k in range(4):
                        lin42 = cbase42 + (jnp.uint32(o + k * 16) + vec_iota)
                        buf[pl.ds(o + k * 16, 16)] = _gumbel_bits(lin42)
                    return 0

                lax.fori_loop(0, chunk // 64, vec_body, 0)
                pltpu.sync_copy(buf, out_hbm.at[row, pl.ds(coff, chunk)])
                return 0

            lax.fori_loop(0, nchunk, chunk_body, 0)

    return gen()


def _tc_main_body(x_ref, ay_ref, ai_ref, as_ref, acc_y, acc_i, acc_s, *,
                  cb, v):
    i = pl.program_id(0)
    g = pl.num_programs(0)
    b = x_ref.shape[0]

    @pl.when(i == 0)
    def _init():
        acc_y[...] = jnp.full_like(acc_y, -jnp.inf)
        acc_i[...] = jnp.zeros_like(acc_i)
        acc_s[...] = jnp.zeros_like(acc_s)

    roff = jax.lax.broadcasted_iota(jnp.int32, (b, 1), 0) * v
    lane = jax.lax.broadcasted_iota(jnp.int32, (b, 128), 1)
    base42 = roff + lane + 42

    ay = acc_y[...]
    ai = acc_i[...]
    asum = acc_s[...]
    for j in range(cb // 128):
        x = x_ref[:, j * 128:(j + 1) * 128]
        off = i * cb + j * 128
        lin42 = base42 + off
        gum = _bits_to_gumbel(_gumbel_bits(lin42.astype(jnp.uint32)))
        y = x + gum
        upd = y > ay
        ay = jnp.where(upd, y, ay)
        ai = jnp.where(upd, lin42, ai)
        asum = asum + jnp.exp(x)
    acc_y[...] = ay
    acc_i[...] = ai
    acc_s[...] = asum

    @pl.when(i == g - 1)
    def _emit():
        ay_ref[...] = acc_y[...]
        ai_ref[...] = acc_i[...]
        as_ref[...] = acc_s[...]


def _tc_tail_body(x_ref, bits_ref, ay0_ref, ai0_ref, as0_ref,
                  samples_ref, lp_ref, acc_y, acc_i, acc_s, *,
                  cb, v, c_start, s):
    i = pl.program_id(0)
    g = pl.num_programs(0)
    b = x_ref.shape[0]

    @pl.when(i == 0)
    def _init():
        acc_y[...] = ay0_ref[...]
        acc_i[...] = ai0_ref[...]
        acc_s[...] = as0_ref[...]

    roff = jax.lax.broadcasted_iota(jnp.int32, (b, 1), 0) * v
    lane = jax.lax.broadcasted_iota(jnp.int32, (b, 128), 1)
    base42 = roff + lane + (c_start + 42)

    def scan_block(masked):
        ay = acc_y[...]
        ai = acc_i[...]
        asum = acc_s[...]
        for j in range(cb // 128):
            x = x_ref[:, j * 128:(j + 1) * 128]
            bits = bits_ref[:, j * 128:(j + 1) * 128]
            off = i * cb + j * 128
            lin42 = base42 + off
            gum = _bits_to_gumbel(bits)
            if masked:
                valid = lane < (s - off)
                y = jnp.where(valid, x + gum, -jnp.inf)
                es = jnp.where(valid, jnp.exp(x), 0.0)
            else:
                y = x + gum
                es = jnp.exp(x)
            upd = y > ay
            ay = jnp.where(upd, y, ay)
            ai = jnp.where(upd, lin42, ai)
            asum = asum + es
        acc_y[...] = ay
        acc_i[...] = ai
        acc_s[...] = asum

    @pl.when(i < g - 1)
    def _full():
        scan_block(False)

    @pl.when(i == g - 1)
    def _ragged():
        scan_block(True)

    @pl.when(i == g - 1)
    def _finish():
        ayf = acc_y[...]
        m = jnp.max(ayf, axis=1, keepdims=True)
        lin42_w = jnp.min(
            jnp.where(ayf == m, acc_i[...], jnp.int32(0x7FFFFFFF)),
            axis=1, keepdims=True)
        gum_w = _gumbel_pre(lin42_w.astype(jnp.uint32))
        lse = jnp.log(jnp.sum(acc_s[...], axis=1, keepdims=True))
        samples_ref[...] = lin42_w - 42 - roff
        lp_ref[...] = (m - gum_w) - lse


def kernel(logits):
    b, v = logits.shape
    cb = 2048
    nmain = max(v // cb - 135, 0)
    c_start = nmain * cb
    s = v - c_start

    bits = _sc_bits_kernel(b, v, c_start, s)

    ay, ai, asum = pl.pallas_call(
        functools.partial(_tc_main_body, cb=cb, v=v),
        grid=(nmain,),
        in_specs=[pl.BlockSpec((b, cb), lambda i: (0, i))],
        out_specs=[pl.BlockSpec((b, 128), lambda i: (0, 0)),
                   pl.BlockSpec((b, 128), lambda i: (0, 0)),
                   pl.BlockSpec((b, 128), lambda i: (0, 0))],
        out_shape=[jax.ShapeDtypeStruct((b, 128), jnp.float32),
                   jax.ShapeDtypeStruct((b, 128), jnp.int32),
                   jax.ShapeDtypeStruct((b, 128), jnp.float32)],
        scratch_shapes=[
            pltpu.VMEM((b, 128), jnp.float32),
            pltpu.VMEM((b, 128), jnp.int32),
            pltpu.VMEM((b, 128), jnp.float32),
        ],
    )(logits)

    samples, lp = pl.pallas_call(
        functools.partial(_tc_tail_body, cb=cb, v=v, c_start=c_start, s=s),
        grid=(pl.cdiv(s, cb),),
        in_specs=[pl.BlockSpec((b, cb), lambda i: (0, i + nmain)),
                  pl.BlockSpec((b, cb), lambda i: (0, i)),
                  pl.BlockSpec((b, 128), lambda i: (0, 0)),
                  pl.BlockSpec((b, 128), lambda i: (0, 0)),
                  pl.BlockSpec((b, 128), lambda i: (0, 0))],
        out_specs=[pl.BlockSpec((b, 1), lambda i: (0, 0)),
                   pl.BlockSpec((b, 1), lambda i: (0, 0))],
        out_shape=[jax.ShapeDtypeStruct((b, 1), jnp.int32),
                   jax.ShapeDtypeStruct((b, 1), jnp.float32)],
        scratch_shapes=[
            pltpu.VMEM((b, 128), jnp.float32),
            pltpu.VMEM((b, 128), jnp.int32),
            pltpu.VMEM((b, 128), jnp.float32),
        ],
    )(logits, bits, ay, ai, asum)
    return samples[:, 0], lp[:, 0]
